# baseline (speedup 1.0000x reference)
import functools

import jax
import jax.numpy as jnp
from jax import lax
from jax.experimental import pallas as pl
from jax.experimental.pallas import tpu as pltpu
from jax.experimental.pallas import tpu_sc as plsc

_K = 8192
_D = 256
_B = 16384
_BETA = 0.25

_BM = 512
_BK = 1024
_BL = 2048

_NC = 2
_NS = 16
_NW = _NC * _NS
_CH = 128


def _argmin_body(enc_ref, emb_ref, encsq_ref, embsq_ref, idx_ref,
                 val_scr, idx_scr):
    j = pl.program_id(1)
    nj = pl.num_programs(1)

    @pl.when(j == 0)
    def _init():
        val_scr[...] = jnp.full((_BM, 1), jnp.inf, jnp.float32)
        idx_scr[...] = jnp.zeros((_BM, 1), jnp.int32)

    s = lax.dot_general(enc_ref[...], emb_ref[...],
                        (((1,), (1,)), ((), ())),
                        preferred_element_type=jnp.float32)
    d2 = (encsq_ref[...] - 2.0 * s) + embsq_ref[...]
    dist = jnp.sqrt(jnp.maximum(d2, 0.0))
    local_min = jnp.min(dist, axis=1, keepdims=True)
    iota = lax.broadcasted_iota(jnp.int32, (_BM, _BK), 1) + j * _BK
    big = jnp.int32(2**31 - 1)
    local_idx = jnp.min(jnp.where(dist == local_min, iota, big),
                        axis=1, keepdims=True)
    take = local_min < val_scr[...]
    idx_scr[...] = jnp.where(take, local_idx, idx_scr[...])
    val_scr[...] = jnp.where(take, local_min, val_scr[...])

    @pl.when(j == nj - 1)
    def _fin():
        idx_ref[...] = idx_scr[...]


def _nearest_index(encoding, embedding, enc_sq, emb_sq):
    return pl.pallas_call(
        _argmin_body,
        grid=(_B // _BM, _K // _BK),
        in_specs=[
            pl.BlockSpec((_BM, _D), lambda i, j: (i, 0)),
            pl.BlockSpec((_BK, _D), lambda i, j: (j, 0)),
            pl.BlockSpec((_BM, 1), lambda i, j: (i, 0)),
            pl.BlockSpec((1, _BK), lambda i, j: (0, j)),
        ],
        out_specs=pl.BlockSpec((_BM, 1), lambda i, j: (i, 0)),
        out_shape=jax.ShapeDtypeStruct((_B, 1), jnp.int32),
        scratch_shapes=[
            pltpu.VMEM((_BM, 1), jnp.float32),
            pltpu.VMEM((_BM, 1), jnp.int32),
        ],
        compiler_params=pltpu.CompilerParams(
            dimension_semantics=("parallel", "arbitrary")),
    )(encoding, embedding, enc_sq, emb_sq)


def _gather_rows(table, idx):
    b_per_w = _B // _NW
    mesh = plsc.VectorSubcoreMesh(core_axis_name="c", subcore_axis_name="s")

    @functools.partial(
        pl.kernel, mesh=mesh,
        out_type=jax.ShapeDtypeStruct((_B, _D), jnp.float32),
        scratch_types=[
            pltpu.VMEM((_CH,), jnp.int32),
            pltpu.VMEM((_CH, _D), jnp.float32),
            pltpu.SemaphoreType.DMA,
        ],
    )
    def _k(table_hbm, idx_hbm, out_hbm, idx_v, rows_v, sem):
        wid = lax.axis_index("s") * _NC + lax.axis_index("c")
        base = wid * b_per_w
        for c in range(b_per_w // _CH):
            off = base + c * _CH
            pltpu.sync_copy(idx_hbm.at[pl.ds(off, _CH)], idx_v)
            pltpu.async_copy(table_hbm.at[idx_v], rows_v, sem).wait()
            pltpu.sync_copy(rows_v, out_hbm.at[pl.ds(off, _CH)])

    return _k(table, idx)


def _st_loss_body(enc_ref, q_ref, st_ref, acc_ref):
    i = pl.program_id(0)
    e = enc_ref[...]
    q = q_ref[...]
    st_ref[...] = e + (q - e)

    @pl.when(i == 0)
    def _init():
        acc_ref[0] = jnp.float32(0.0)

    acc_ref[0] += jnp.sum((e - q) ** 2)


def _st_and_loss(encoding, quantized):
    return pl.pallas_call(
        _st_loss_body,
        grid=(_B // _BL,),
        in_specs=[
            pl.BlockSpec((_BL, _D), lambda i: (i, 0)),
            pl.BlockSpec((_BL, _D), lambda i: (i, 0)),
        ],
        out_specs=[
            pl.BlockSpec((_BL, _D), lambda i: (i, 0)),
            pl.BlockSpec(memory_space=pltpu.SMEM),
        ],
        out_shape=[
            jax.ShapeDtypeStruct((_B, _D), jnp.float32),
            jax.ShapeDtypeStruct((1,), jnp.float32),
        ],
        compiler_params=pltpu.CompilerParams(
            dimension_semantics=("arbitrary",)),
    )(encoding, quantized)


def kernel(encoding, embedding):
    enc_sq = jnp.sum(encoding ** 2, axis=1, keepdims=True)
    emb_sq = jnp.sum(embedding ** 2, axis=1)[None, :]
    idx2 = _nearest_index(encoding, embedding, enc_sq, emb_sq)
    quantized_index = idx2[:, 0]
    quantized_embedding = _gather_rows(embedding, quantized_index)
    quantized_st, sq_sum = _st_and_loss(encoding, quantized_embedding)
    loss = sq_sum[0] / jnp.float32(_B * _D)
    commitment_loss = loss
    embedding_loss = loss
    vq_loss = commitment_loss * _BETA + embedding_loss
    return (quantized_index, quantized_st, vq_loss, embedding_loss,
            commitment_loss)


# speedup vs baseline: 1.2780x; 1.2780x over previous
import functools

import jax
import jax.numpy as jnp
from jax import lax
from jax.experimental import pallas as pl
from jax.experimental.pallas import tpu as pltpu
from jax.experimental.pallas import tpu_sc as plsc

_K = 8192
_D = 256
_B = 16384
_BETA = 0.25

_BM = 1024
_BK = 2048
_GC = _BK // 128
_BL = 2048

_NC = 2
_NS = 16
_NW = _NC * _NS
_CH = 128


def _argmin_body(enc_ref, emb_ref, encsq_ref, embsq_ref, idx_ref,
                 val_scr, idx_scr):
    j = pl.program_id(1)
    nj = pl.num_programs(1)

    @pl.when(j == 0)
    def _init():
        val_scr[...] = jnp.full((_BM, 128), jnp.inf, jnp.float32)
        idx_scr[...] = jnp.zeros((_BM, 128), jnp.int32)

    s = lax.dot_general(enc_ref[...], emb_ref[...],
                        (((1,), (1,)), ((), ())),
                        preferred_element_type=jnp.float32)
    encsq = encsq_ref[...]
    embsq = embsq_ref[...]
    val = val_scr[...]
    idxc = idx_scr[...]
    for g in range(_GC):
        d2 = (encsq - 2.0 * s[:, g * 128:(g + 1) * 128]) \
            + embsq[:, g * 128:(g + 1) * 128]
        dist = jnp.sqrt(jnp.maximum(d2, 0.0))
        lt = dist < val
        idxc = jnp.where(lt, j * _GC + g, idxc)
        val = jnp.where(lt, dist, val)
    val_scr[...] = val
    idx_scr[...] = idxc

    @pl.when(j == nj - 1)
    def _fin():
        lane = lax.broadcasted_iota(jnp.int32, (_BM, 128), 1)
        full = idx_scr[...] * 128 + lane
        v = val_scr[...]
        m = jnp.min(v, axis=1, keepdims=True)
        big = jnp.int32(2**31 - 1)
        cand = jnp.where(v == m, full, big)
        idx_ref[...] = jnp.min(cand, axis=1, keepdims=True)


def _nearest_index(encoding, embedding, enc_sq, emb_sq):
    return pl.pallas_call(
        _argmin_body,
        grid=(_B // _BM, _K // _BK),
        in_specs=[
            pl.BlockSpec((_BM, _D), lambda i, j: (i, 0)),
            pl.BlockSpec((_BK, _D), lambda i, j: (j, 0)),
            pl.BlockSpec((_BM, 1), lambda i, j: (i, 0)),
            pl.BlockSpec((1, _BK), lambda i, j: (0, j)),
        ],
        out_specs=pl.BlockSpec((_BM, 1), lambda i, j: (i, 0)),
        out_shape=jax.ShapeDtypeStruct((_B, 1), jnp.int32),
        scratch_shapes=[
            pltpu.VMEM((_BM, 128), jnp.float32),
            pltpu.VMEM((_BM, 128), jnp.int32),
        ],
        compiler_params=pltpu.CompilerParams(
            dimension_semantics=("parallel", "arbitrary")),
    )(encoding, embedding, enc_sq, emb_sq)


def _gather_rows(table, idx):
    b_per_w = _B // _NW
    mesh = plsc.VectorSubcoreMesh(core_axis_name="c", subcore_axis_name="s")

    @functools.partial(
        pl.kernel, mesh=mesh,
        out_type=jax.ShapeDtypeStruct((_B, _D), jnp.float32),
        scratch_types=[
            pltpu.VMEM((_CH,), jnp.int32),
            pltpu.VMEM((_CH, _D), jnp.float32),
            pltpu.SemaphoreType.DMA,
        ],
    )
    def _k(table_hbm, idx_hbm, out_hbm, idx_v, rows_v, sem):
        wid = lax.axis_index("s") * _NC + lax.axis_index("c")
        base = wid * b_per_w
        for c in range(b_per_w // _CH):
            off = base + c * _CH
            pltpu.sync_copy(idx_hbm.at[pl.ds(off, _CH)], idx_v)
            pltpu.async_copy(table_hbm.at[idx_v], rows_v, sem).wait()
            pltpu.sync_copy(rows_v, out_hbm.at[pl.ds(off, _CH)])

    return _k(table, idx)


def _st_loss_body(enc_ref, q_ref, st_ref, acc_ref):
    i = pl.program_id(0)
    e = enc_ref[...]
    q = q_ref[...]
    st_ref[...] = e + (q - e)

    @pl.when(i == 0)
    def _init():
        acc_ref[0] = jnp.float32(0.0)

    acc_ref[0] += jnp.sum((e - q) ** 2)


def _st_and_loss(encoding, quantized):
    return pl.pallas_call(
        _st_loss_body,
        grid=(_B // _BL,),
        in_specs=[
            pl.BlockSpec((_BL, _D), lambda i: (i, 0)),
            pl.BlockSpec((_BL, _D), lambda i: (i, 0)),
        ],
        out_specs=[
            pl.BlockSpec((_BL, _D), lambda i: (i, 0)),
            pl.BlockSpec(memory_space=pltpu.SMEM),
        ],
        out_shape=[
            jax.ShapeDtypeStruct((_B, _D), jnp.float32),
            jax.ShapeDtypeStruct((1,), jnp.float32),
        ],
        compiler_params=pltpu.CompilerParams(
            dimension_semantics=("arbitrary",)),
    )(encoding, quantized)


def kernel(encoding, embedding):
    enc_sq = jnp.sum(encoding ** 2, axis=1, keepdims=True)
    emb_sq = jnp.sum(embedding ** 2, axis=1)[None, :]
    idx2 = _nearest_index(encoding, embedding, enc_sq, emb_sq)
    quantized_index = idx2[:, 0]
    quantized_embedding = _gather_rows(embedding, quantized_index)
    quantized_st, sq_sum = _st_and_loss(encoding, quantized_embedding)
    loss = sq_sum[0] / jnp.float32(_B * _D)
    commitment_loss = loss
    embedding_loss = loss
    vq_loss = commitment_loss * _BETA + embedding_loss
    return (quantized_index, quantized_st, vq_loss, embedding_loss,
            commitment_loss)


# speedup vs baseline: 1.6516x; 1.2923x over previous
import functools

import jax
import jax.numpy as jnp
from jax import lax
from jax.experimental import pallas as pl
from jax.experimental.pallas import tpu as pltpu
from jax.experimental.pallas import tpu_sc as plsc

_K = 8192
_D = 256
_B = 16384
_BETA = 0.25

_BM = 1024
_BK = 4096
_GC = _BK // 128
_RB = 128
_BL = 2048

_NC = 2
_NS = 16
_NW = _NC * _NS
_CH = 128


def _argmin_body(enc_ref, emb2_ref, encsq_ref, embsq_ref, idx_ref,
                 val_scr, idx_scr):
    j = pl.program_id(1)
    nj = pl.num_programs(1)

    @pl.when(j == 0)
    def _init():
        val_scr[...] = jnp.full((_BM, 128), jnp.inf, jnp.float32)
        idx_scr[...] = jnp.zeros((_BM, 128), jnp.int32)

    s2 = lax.dot_general(enc_ref[...], emb2_ref[...],
                         (((1,), (1,)), ((), ())),
                         preferred_element_type=jnp.float32)
    embsq = embsq_ref[...]
    for r in range(_BM // _RB):
        rs = slice(r * _RB, (r + 1) * _RB)
        encb = jnp.broadcast_to(encsq_ref[rs], (_RB, 128))
        val = val_scr[rs]
        idxc = idx_scr[rs]
        for g in range(_GC):
            cs = slice(g * 128, (g + 1) * 128)
            d2 = (encb - s2[rs, cs]) + embsq[:, cs]
            mx = jnp.maximum(d2, 0.0)
            dist = mx * lax.rsqrt(mx)
            lt = dist < val
            idxc = jnp.where(lt, j * _GC + g, idxc)
            val = jnp.where(lt, dist, val)
        val_scr[rs] = val
        idx_scr[rs] = idxc

    @pl.when(j == nj - 1)
    def _fin():
        lane = lax.broadcasted_iota(jnp.int32, (_BM, 128), 1)
        full = idx_scr[...] * 128 + lane
        v = val_scr[...]
        m = jnp.min(v, axis=1, keepdims=True)
        big = jnp.int32(2**31 - 1)
        cand = jnp.where(v == m, full, big)
        idx_ref[...] = jnp.min(cand, axis=1, keepdims=True)


def _nearest_index(encoding, embedding, enc_sq, emb_sq):
    return pl.pallas_call(
        _argmin_body,
        grid=(_B // _BM, _K // _BK),
        in_specs=[
            pl.BlockSpec((_BM, _D), lambda i, j: (i, 0)),
            pl.BlockSpec((_BK, _D), lambda i, j: (j, 0)),
            pl.BlockSpec((_BM, 1), lambda i, j: (i, 0)),
            pl.BlockSpec((1, _BK), lambda i, j: (0, j)),
        ],
        out_specs=pl.BlockSpec((_BM, 1), lambda i, j: (i, 0)),
        out_shape=jax.ShapeDtypeStruct((_B, 1), jnp.int32),
        scratch_shapes=[
            pltpu.VMEM((_BM, 128), jnp.float32),
            pltpu.VMEM((_BM, 128), jnp.int32),
        ],
        compiler_params=pltpu.CompilerParams(
            dimension_semantics=("parallel", "arbitrary")),
    )(encoding, embedding, enc_sq, emb_sq)


def _gather_rows(table, idx):
    b_per_w = _B // _NW
    mesh = plsc.VectorSubcoreMesh(core_axis_name="c", subcore_axis_name="s")

    @functools.partial(
        pl.kernel, mesh=mesh,
        out_type=jax.ShapeDtypeStruct((_B, _D), jnp.float32),
        scratch_types=[
            pltpu.VMEM((_CH,), jnp.int32),
            pltpu.VMEM((_CH,), jnp.int32),
            pltpu.VMEM((_CH, _D), jnp.float32),
            pltpu.VMEM((_CH, _D), jnp.float32),
            pltpu.SemaphoreType.DMA,
            pltpu.SemaphoreType.DMA,
        ],
    )
    def _k(table_hbm, idx_hbm, out_hbm, idx0, idx1, rows0, rows1, s0, s1):
        wid = lax.axis_index("s") * _NC + lax.axis_index("c")
        base = wid * b_per_w
        bufs = ((idx0, rows0, s0), (idx1, rows1, s1))
        n = b_per_w // _CH
        pltpu.sync_copy(idx_hbm.at[pl.ds(base, _CH)], idx0)
        cps = [pltpu.async_copy(table_hbm.at[idx0], rows0, s0)]
        for c in range(n):
            if c + 1 < n:
                ib, rb, sb = bufs[(c + 1) % 2]
                off = base + (c + 1) * _CH
                pltpu.sync_copy(idx_hbm.at[pl.ds(off, _CH)], ib)
                cps.append(pltpu.async_copy(table_hbm.at[ib], rb, sb))
            cps[c].wait()
            pltpu.sync_copy(bufs[c % 2][1],
                            out_hbm.at[pl.ds(base + c * _CH, _CH)])

    return _k(table, idx)


def _st_loss_body(enc_ref, q_ref, st_ref, acc_ref):
    i = pl.program_id(0)
    e = enc_ref[...]
    q = q_ref[...]
    st_ref[...] = e + (q - e)

    @pl.when(i == 0)
    def _init():
        acc_ref[0] = jnp.float32(0.0)

    acc_ref[0] += jnp.sum((e - q) ** 2)


def _st_and_loss(encoding, quantized):
    return pl.pallas_call(
        _st_loss_body,
        grid=(_B // _BL,),
        in_specs=[
            pl.BlockSpec((_BL, _D), lambda i: (i, 0)),
            pl.BlockSpec((_BL, _D), lambda i: (i, 0)),
        ],
        out_specs=[
            pl.BlockSpec((_BL, _D), lambda i: (i, 0)),
            pl.BlockSpec(memory_space=pltpu.SMEM),
        ],
        out_shape=[
            jax.ShapeDtypeStruct((_B, _D), jnp.float32),
            jax.ShapeDtypeStruct((1,), jnp.float32),
        ],
        compiler_params=pltpu.CompilerParams(
            dimension_semantics=("arbitrary",)),
    )(encoding, quantized)


def kernel(encoding, embedding):
    enc_sq = jnp.sum(encoding ** 2, axis=1, keepdims=True)
    emb_sq = jnp.sum(embedding ** 2, axis=1)[None, :]
    emb2 = embedding + embedding
    idx2 = _nearest_index(encoding, emb2, enc_sq, emb_sq)
    quantized_index = idx2[:, 0]
    quantized_embedding = _gather_rows(embedding, quantized_index)
    quantized_st, sq_sum = _st_and_loss(encoding, quantized_embedding)
    loss = sq_sum[0] / jnp.float32(_B * _D)
    commitment_loss = loss
    embedding_loss = loss
    vq_loss = commitment_loss * _BETA + embedding_loss
    return (quantized_index, quantized_st, vq_loss, embedding_loss,
            commitment_loss)


# speedup vs baseline: 1.7718x; 1.0728x over previous
import functools

import jax
import jax.numpy as jnp
from jax import lax
from jax.experimental import pallas as pl
from jax.experimental.pallas import tpu as pltpu
from jax.experimental.pallas import tpu_sc as plsc

_K = 8192
_D = 256
_B = 16384
_BETA = 0.25

_BM = 2048
_BK = 4096
_GC = _BK // 128
_RB = 128
_BL = 2048

_NC = 2
_NS = 16
_NW = _NC * _NS
_CH = 128


def _argmin_body(enc_ref, emb2_ref, encsq_ref, embsq_ref, idx_ref,
                 val_scr, idx_scr):
    j = pl.program_id(1)
    nj = pl.num_programs(1)

    @pl.when(j == 0)
    def _init():
        val_scr[...] = jnp.full((_BM, 128), jnp.inf, jnp.float32)
        idx_scr[...] = jnp.zeros((_BM, 128), jnp.int32)

    s2 = lax.dot_general(enc_ref[...], emb2_ref[...],
                         (((1,), (1,)), ((), ())),
                         preferred_element_type=jnp.float32)
    embsq = embsq_ref[...]
    for r in range(_BM // _RB):
        rs = slice(r * _RB, (r + 1) * _RB)
        encb = jnp.broadcast_to(encsq_ref[rs], (_RB, 128))
        val = val_scr[rs]
        idxc = idx_scr[rs]
        for g in range(_GC):
            cs = slice(g * 128, (g + 1) * 128)
            d2 = (encb - s2[rs, cs]) + embsq[:, cs]
            mx = jnp.maximum(d2, 0.0)
            dist = mx * lax.rsqrt(mx)
            lt = dist < val
            idxc = jnp.where(lt, j * _GC + g, idxc)
            val = jnp.where(lt, dist, val)
        val_scr[rs] = val
        idx_scr[rs] = idxc

    @pl.when(j == nj - 1)
    def _fin():
        lane = lax.broadcasted_iota(jnp.int32, (_BM, 128), 1)
        full = idx_scr[...] * 128 + lane
        v = val_scr[...]
        m = jnp.min(v, axis=1, keepdims=True)
        big = jnp.int32(2**31 - 1)
        cand = jnp.where(v == m, full, big)
        idx_ref[...] = jnp.min(cand, axis=1, keepdims=True)


def _nearest_index(encoding, embedding, enc_sq, emb_sq):
    return pl.pallas_call(
        _argmin_body,
        grid=(_B // _BM, _K // _BK),
        in_specs=[
            pl.BlockSpec((_BM, _D), lambda i, j: (i, 0)),
            pl.BlockSpec((_BK, _D), lambda i, j: (j, 0)),
            pl.BlockSpec((_BM, 1), lambda i, j: (i, 0)),
            pl.BlockSpec((1, _BK), lambda i, j: (0, j)),
        ],
        out_specs=pl.BlockSpec((_BM, 1), lambda i, j: (i, 0)),
        out_shape=jax.ShapeDtypeStruct((_B, 1), jnp.int32),
        scratch_shapes=[
            pltpu.VMEM((_BM, 128), jnp.float32),
            pltpu.VMEM((_BM, 128), jnp.int32),
        ],
        compiler_params=pltpu.CompilerParams(
            dimension_semantics=("parallel", "arbitrary")),
    )(encoding, embedding, enc_sq, emb_sq)


def _gather_rows(table, idx):
    b_per_w = _B // _NW
    mesh = plsc.VectorSubcoreMesh(core_axis_name="c", subcore_axis_name="s")

    @functools.partial(
        pl.kernel, mesh=mesh,
        out_type=jax.ShapeDtypeStruct((_B, _D), jnp.float32),
        scratch_types=[
            pltpu.VMEM((_CH,), jnp.int32),
            pltpu.VMEM((_CH,), jnp.int32),
            pltpu.VMEM((_CH, _D), jnp.float32),
            pltpu.VMEM((_CH, _D), jnp.float32),
            pltpu.SemaphoreType.DMA,
            pltpu.SemaphoreType.DMA,
        ],
    )
    def _k(table_hbm, idx_hbm, out_hbm, idx0, idx1, rows0, rows1, s0, s1):
        wid = lax.axis_index("s") * _NC + lax.axis_index("c")
        base = wid * b_per_w
        bufs = ((idx0, rows0, s0), (idx1, rows1, s1))
        n = b_per_w // _CH
        pltpu.sync_copy(idx_hbm.at[pl.ds(base, _CH)], idx0)
        cps = [pltpu.async_copy(table_hbm.at[idx0], rows0, s0)]
        for c in range(n):
            if c + 1 < n:
                ib, rb, sb = bufs[(c + 1) % 2]
                off = base + (c + 1) * _CH
                pltpu.sync_copy(idx_hbm.at[pl.ds(off, _CH)], ib)
                cps.append(pltpu.async_copy(table_hbm.at[ib], rb, sb))
            cps[c].wait()
            pltpu.sync_copy(bufs[c % 2][1],
                            out_hbm.at[pl.ds(base + c * _CH, _CH)])

    return _k(table, idx)


def _st_loss_body(enc_ref, q_ref, st_ref, acc_ref):
    i = pl.program_id(0)
    e = enc_ref[...]
    q = q_ref[...]
    st_ref[...] = e + (q - e)

    @pl.when(i == 0)
    def _init():
        acc_ref[0] = jnp.float32(0.0)

    acc_ref[0] += jnp.sum((e - q) ** 2)


def _st_and_loss(encoding, quantized):
    return pl.pallas_call(
        _st_loss_body,
        grid=(_B // _BL,),
        in_specs=[
            pl.BlockSpec((_BL, _D), lambda i: (i, 0)),
            pl.BlockSpec((_BL, _D), lambda i: (i, 0)),
        ],
        out_specs=[
            pl.BlockSpec((_BL, _D), lambda i: (i, 0)),
            pl.BlockSpec(memory_space=pltpu.SMEM),
        ],
        out_shape=[
            jax.ShapeDtypeStruct((_B, _D), jnp.float32),
            jax.ShapeDtypeStruct((1,), jnp.float32),
        ],
        compiler_params=pltpu.CompilerParams(
            dimension_semantics=("arbitrary",)),
    )(encoding, quantized)


def kernel(encoding, embedding):
    enc_sq = jnp.sum(encoding ** 2, axis=1, keepdims=True)
    emb_sq = jnp.sum(embedding ** 2, axis=1)[None, :]
    emb2 = embedding + embedding
    idx2 = _nearest_index(encoding, emb2, enc_sq, emb_sq)
    quantized_index = idx2[:, 0]
    quantized_embedding = _gather_rows(embedding, quantized_index)
    quantized_st, sq_sum = _st_and_loss(encoding, quantized_embedding)
    loss = sq_sum[0] / jnp.float32(_B * _D)
    commitment_loss = loss
    embedding_loss = loss
    vq_loss = commitment_loss * _BETA + embedding_loss
    return (quantized_index, quantized_st, vq_loss, embedding_loss,
            commitment_loss)


# speedup vs baseline: 1.8416x; 1.0394x over previous
import functools

import jax
import jax.numpy as jnp
from jax import lax
from jax.experimental import pallas as pl
from jax.experimental.pallas import tpu as pltpu
from jax.experimental.pallas import tpu_sc as plsc

_K = 8192
_D = 256
_B = 16384
_BETA = 0.25

_BM = 4096
_BK = 2048
_GC = _BK // 128
_RB = 128
_BL = 2048

_NC = 2
_NS = 16
_NW = _NC * _NS
_CH = 128


def _argmin_body(enc_ref, emb2_ref, encsq_ref, embsq_ref, idx_ref,
                 val_scr, idx_scr):
    j = pl.program_id(1)
    nj = pl.num_programs(1)

    @pl.when(j == 0)
    def _init():
        val_scr[...] = jnp.full((_BM, 128), jnp.inf, jnp.float32)
        idx_scr[...] = jnp.zeros((_BM, 128), jnp.int32)

    s2 = lax.dot_general(enc_ref[...], emb2_ref[...],
                         (((1,), (1,)), ((), ())),
                         preferred_element_type=jnp.float32)
    embsq = embsq_ref[...]
    for r in range(_BM // _RB):
        rs = slice(r * _RB, (r + 1) * _RB)
        encb = jnp.broadcast_to(encsq_ref[rs], (_RB, 128))
        val = val_scr[rs]
        idxc = idx_scr[rs]
        for g in range(_GC):
            cs = slice(g * 128, (g + 1) * 128)
            d2 = (encb - s2[rs, cs]) + embsq[:, cs]
            mx = jnp.maximum(d2, 0.0)
            dist = mx * lax.rsqrt(mx)
            lt = dist < val
            idxc = jnp.where(lt, j * _GC + g, idxc)
            val = jnp.where(lt, dist, val)
        val_scr[rs] = val
        idx_scr[rs] = idxc

    @pl.when(j == nj - 1)
    def _fin():
        lane = lax.broadcasted_iota(jnp.int32, (_BM, 128), 1)
        full = idx_scr[...] * 128 + lane
        v = val_scr[...]
        m = jnp.min(v, axis=1, keepdims=True)
        big = jnp.int32(2**31 - 1)
        cand = jnp.where(v == m, full, big)
        idx_ref[...] = jnp.min(cand, axis=1, keepdims=True)


def _nearest_index(encoding, embedding, enc_sq, emb_sq):
    return pl.pallas_call(
        _argmin_body,
        grid=(_B // _BM, _K // _BK),
        in_specs=[
            pl.BlockSpec((_BM, _D), lambda i, j: (i, 0)),
            pl.BlockSpec((_BK, _D), lambda i, j: (j, 0)),
            pl.BlockSpec((_BM, 1), lambda i, j: (i, 0)),
            pl.BlockSpec((1, _BK), lambda i, j: (0, j)),
        ],
        out_specs=pl.BlockSpec((_BM, 1), lambda i, j: (i, 0)),
        out_shape=jax.ShapeDtypeStruct((_B, 1), jnp.int32),
        scratch_shapes=[
            pltpu.VMEM((_BM, 128), jnp.float32),
            pltpu.VMEM((_BM, 128), jnp.int32),
        ],
        compiler_params=pltpu.CompilerParams(
            dimension_semantics=("parallel", "arbitrary")),
    )(encoding, embedding, enc_sq, emb_sq)


def _gather_rows(table, idx):
    b_per_w = _B // _NW
    mesh = plsc.VectorSubcoreMesh(core_axis_name="c", subcore_axis_name="s")

    @functools.partial(
        pl.kernel, mesh=mesh,
        out_type=jax.ShapeDtypeStruct((_B, _D), jnp.float32),
        scratch_types=[
            pltpu.VMEM((_CH,), jnp.int32),
            pltpu.VMEM((_CH,), jnp.int32),
            pltpu.VMEM((_CH, _D), jnp.float32),
            pltpu.VMEM((_CH, _D), jnp.float32),
            pltpu.SemaphoreType.DMA,
            pltpu.SemaphoreType.DMA,
        ],
    )
    def _k(table_hbm, idx_hbm, out_hbm, idx0, idx1, rows0, rows1, s0, s1):
        wid = lax.axis_index("s") * _NC + lax.axis_index("c")
        base = wid * b_per_w
        bufs = ((idx0, rows0, s0), (idx1, rows1, s1))
        n = b_per_w // _CH
        pltpu.sync_copy(idx_hbm.at[pl.ds(base, _CH)], idx0)
        cps = [pltpu.async_copy(table_hbm.at[idx0], rows0, s0)]
        for c in range(n):
            if c + 1 < n:
                ib, rb, sb = bufs[(c + 1) % 2]
                off = base + (c + 1) * _CH
                pltpu.sync_copy(idx_hbm.at[pl.ds(off, _CH)], ib)
                cps.append(pltpu.async_copy(table_hbm.at[ib], rb, sb))
            cps[c].wait()
            pltpu.sync_copy(bufs[c % 2][1],
                            out_hbm.at[pl.ds(base + c * _CH, _CH)])

    return _k(table, idx)


def _st_loss_body(enc_ref, q_ref, st_ref, acc_ref):
    i = pl.program_id(0)
    e = enc_ref[...]
    q = q_ref[...]
    st_ref[...] = e + (q - e)

    @pl.when(i == 0)
    def _init():
        acc_ref[0] = jnp.float32(0.0)

    acc_ref[0] += jnp.sum((e - q) ** 2)


def _st_and_loss(encoding, quantized):
    return pl.pallas_call(
        _st_loss_body,
        grid=(_B // _BL,),
        in_specs=[
            pl.BlockSpec((_BL, _D), lambda i: (i, 0)),
            pl.BlockSpec((_BL, _D), lambda i: (i, 0)),
        ],
        out_specs=[
            pl.BlockSpec((_BL, _D), lambda i: (i, 0)),
            pl.BlockSpec(memory_space=pltpu.SMEM),
        ],
        out_shape=[
            jax.ShapeDtypeStruct((_B, _D), jnp.float32),
            jax.ShapeDtypeStruct((1,), jnp.float32),
        ],
        compiler_params=pltpu.CompilerParams(
            dimension_semantics=("arbitrary",)),
    )(encoding, quantized)


def kernel(encoding, embedding):
    enc_sq = jnp.sum(encoding ** 2, axis=1, keepdims=True)
    emb_sq = jnp.sum(embedding ** 2, axis=1)[None, :]
    emb2 = embedding + embedding
    idx2 = _nearest_index(encoding, emb2, enc_sq, emb_sq)
    quantized_index = idx2[:, 0]
    quantized_embedding = _gather_rows(embedding, quantized_index)
    quantized_st, sq_sum = _st_and_loss(encoding, quantized_embedding)
    loss = sq_sum[0] / jnp.float32(_B * _D)
    commitment_loss = loss
    embedding_loss = loss
    vq_loss = commitment_loss * _BETA + embedding_loss
    return (quantized_index, quantized_st, vq_loss, embedding_loss,
            commitment_loss)


# speedup vs baseline: 1.8613x; 1.0107x over previous
import functools

import jax
import jax.numpy as jnp
from jax import lax
from jax.experimental import pallas as pl
from jax.experimental.pallas import tpu as pltpu
from jax.experimental.pallas import tpu_sc as plsc

_K = 8192
_D = 256
_B = 16384
_BETA = 0.25

_BM = 8192
_BK = 512
_GC = _BK // 128
_RB = 128
_BL = 2048

_NC = 2
_NS = 16
_NW = _NC * _NS
_CH = 128


def _argmin_body(enc_ref, emb2_ref, encsq_ref, embsq_ref, idx_ref,
                 val_scr, idx_scr):
    j = pl.program_id(1)
    nj = pl.num_programs(1)

    @pl.when(j == 0)
    def _init():
        val_scr[...] = jnp.full((_BM, 128), jnp.inf, jnp.float32)
        idx_scr[...] = jnp.zeros((_BM, 128), jnp.int32)

    s2 = lax.dot_general(enc_ref[...], emb2_ref[...],
                         (((1,), (1,)), ((), ())),
                         preferred_element_type=jnp.float32)
    embsq = embsq_ref[...]
    for r in range(_BM // _RB):
        rs = slice(r * _RB, (r + 1) * _RB)
        encb = jnp.broadcast_to(encsq_ref[rs], (_RB, 128))
        val = val_scr[rs]
        idxc = idx_scr[rs]
        for g in range(_GC):
            cs = slice(g * 128, (g + 1) * 128)
            d2 = (encb - s2[rs, cs]) + embsq[:, cs]
            mx = jnp.maximum(d2, 0.0)
            dist = mx * lax.rsqrt(mx)
            lt = dist < val
            idxc = jnp.where(lt, j * _GC + g, idxc)
            val = jnp.where(lt, dist, val)
        val_scr[rs] = val
        idx_scr[rs] = idxc

    @pl.when(j == nj - 1)
    def _fin():
        lane = lax.broadcasted_iota(jnp.int32, (_BM, 128), 1)
        full = idx_scr[...] * 128 + lane
        v = val_scr[...]
        m = jnp.min(v, axis=1, keepdims=True)
        big = jnp.int32(2**31 - 1)
        cand = jnp.where(v == m, full, big)
        idx_ref[...] = jnp.min(cand, axis=1, keepdims=True)


def _nearest_index(encoding, embedding, enc_sq, emb_sq):
    return pl.pallas_call(
        _argmin_body,
        grid=(_B // _BM, _K // _BK),
        in_specs=[
            pl.BlockSpec((_BM, _D), lambda i, j: (i, 0)),
            pl.BlockSpec((_BK, _D), lambda i, j: (j, 0)),
            pl.BlockSpec((_BM, 1), lambda i, j: (i, 0)),
            pl.BlockSpec((1, _BK), lambda i, j: (0, j)),
        ],
        out_specs=pl.BlockSpec((_BM, 1), lambda i, j: (i, 0)),
        out_shape=jax.ShapeDtypeStruct((_B, 1), jnp.int32),
        scratch_shapes=[
            pltpu.VMEM((_BM, 128), jnp.float32),
            pltpu.VMEM((_BM, 128), jnp.int32),
        ],
        compiler_params=pltpu.CompilerParams(
            dimension_semantics=("parallel", "arbitrary")),
    )(encoding, embedding, enc_sq, emb_sq)


def _gather_rows(table, idx):
    b_per_w = _B // _NW
    mesh = plsc.VectorSubcoreMesh(core_axis_name="c", subcore_axis_name="s")

    @functools.partial(
        pl.kernel, mesh=mesh,
        out_type=jax.ShapeDtypeStruct((_B, _D), jnp.float32),
        scratch_types=[
            pltpu.VMEM((_CH,), jnp.int32),
            pltpu.VMEM((_CH,), jnp.int32),
            pltpu.VMEM((_CH, _D), jnp.float32),
            pltpu.VMEM((_CH, _D), jnp.float32),
            pltpu.SemaphoreType.DMA,
            pltpu.SemaphoreType.DMA,
        ],
    )
    def _k(table_hbm, idx_hbm, out_hbm, idx0, idx1, rows0, rows1, s0, s1):
        wid = lax.axis_index("s") * _NC + lax.axis_index("c")
        base = wid * b_per_w
        bufs = ((idx0, rows0, s0), (idx1, rows1, s1))
        n = b_per_w // _CH
        pltpu.sync_copy(idx_hbm.at[pl.ds(base, _CH)], idx0)
        cps = [pltpu.async_copy(table_hbm.at[idx0], rows0, s0)]
        for c in range(n):
            if c + 1 < n:
                ib, rb, sb = bufs[(c + 1) % 2]
                off = base + (c + 1) * _CH
                pltpu.sync_copy(idx_hbm.at[pl.ds(off, _CH)], ib)
                cps.append(pltpu.async_copy(table_hbm.at[ib], rb, sb))
            cps[c].wait()
            pltpu.sync_copy(bufs[c % 2][1],
                            out_hbm.at[pl.ds(base + c * _CH, _CH)])

    return _k(table, idx)


def _st_loss_body(enc_ref, q_ref, st_ref, acc_ref):
    i = pl.program_id(0)
    e = enc_ref[...]
    q = q_ref[...]
    st_ref[...] = e + (q - e)

    @pl.when(i == 0)
    def _init():
        acc_ref[0] = jnp.float32(0.0)

    acc_ref[0] += jnp.sum((e - q) ** 2)


def _st_and_loss(encoding, quantized):
    return pl.pallas_call(
        _st_loss_body,
        grid=(_B // _BL,),
        in_specs=[
            pl.BlockSpec((_BL, _D), lambda i: (i, 0)),
            pl.BlockSpec((_BL, _D), lambda i: (i, 0)),
        ],
        out_specs=[
            pl.BlockSpec((_BL, _D), lambda i: (i, 0)),
            pl.BlockSpec(memory_space=pltpu.SMEM),
        ],
        out_shape=[
            jax.ShapeDtypeStruct((_B, _D), jnp.float32),
            jax.ShapeDtypeStruct((1,), jnp.float32),
        ],
        compiler_params=pltpu.CompilerParams(
            dimension_semantics=("arbitrary",)),
    )(encoding, quantized)


def kernel(encoding, embedding):
    enc_sq = jnp.sum(encoding ** 2, axis=1, keepdims=True)
    emb_sq = jnp.sum(embedding ** 2, axis=1)[None, :]
    emb2 = embedding + embedding
    idx2 = _nearest_index(encoding, emb2, enc_sq, emb_sq)
    quantized_index = idx2[:, 0]
    quantized_embedding = _gather_rows(embedding, quantized_index)
    quantized_st, sq_sum = _st_and_loss(encoding, quantized_embedding)
    loss = sq_sum[0] / jnp.float32(_B * _D)
    commitment_loss = loss
    embedding_loss = loss
    vq_loss = commitment_loss * _BETA + embedding_loss
    return (quantized_index, quantized_st, vq_loss, embedding_loss,
            commitment_loss)
